# Initial kernel scaffold; baseline (speedup 1.0000x reference)
import jax
import jax.numpy as jnp
from jax.experimental import pallas as pl


def kernel(x, token_table, pos_table):
    raise NotImplementedError("write your pallas kernel here")


# speedup vs baseline: 1.9559x; 1.9559x over previous
import functools

import jax
import jax.numpy as jnp
from jax import lax
from jax.experimental import pallas as pl
from jax.experimental.pallas import tpu as pltpu
from jax.experimental.pallas import tpu_sc as plsc

B, L, V, D = 1024, 200, 100000, 128
NC, NS, LANES = 2, 16, 16
NW = NC * NS
SEQ_PER_W = B // NW
VECS_PER_ROW = D // LANES


def _sc_body(x_hbm, tok_hbm, pos_hbm, out_hbm, idx_v, rows_v, pos_v, sem):
    wid = lax.axis_index("s") * NC + lax.axis_index("c")

    pltpu.sync_copy(pos_hbm, pos_v)

    def seq_body(i, carry):
        b = wid * SEQ_PER_W + i
        pltpu.sync_copy(x_hbm.at[b], idx_v)
        pltpu.async_copy(
            tok_hbm.at[idx_v.at[pl.ds(0, 128)]], rows_v.at[pl.ds(0, 128)], sem
        ).wait()
        pltpu.async_copy(
            tok_hbm.at[idx_v.at[pl.ds(128, 72)]], rows_v.at[pl.ds(128, 72)], sem
        ).wait()

        def add_body(r, c2):
            for c in range(VECS_PER_ROW):
                sl = pl.ds(c * LANES, LANES)
                rows_v[r, sl] = rows_v[r, sl] + pos_v[r, sl]
            return c2

        lax.fori_loop(0, L, add_body, 0, unroll=2)

        pltpu.sync_copy(rows_v, out_hbm.at[b])
        return carry

    lax.fori_loop(0, SEQ_PER_W, seq_body, 0)


@functools.partial(jax.jit, static_argnames=())
def _run(x, token_table, pos_table):
    mesh = plsc.VectorSubcoreMesh(core_axis_name="c", subcore_axis_name="s")
    kfn = functools.partial(
        pl.kernel,
        mesh=mesh,
        out_type=jax.ShapeDtypeStruct((B, L, D), jnp.float32),
        scratch_types=[
            pltpu.VMEM((L,), jnp.int32),
            pltpu.VMEM((L, D), jnp.float32),
            pltpu.VMEM((L, D), jnp.float32),
            pltpu.SemaphoreType.DMA,
        ],
    )(_sc_body)
    return kfn(x, token_table, pos_table)


def kernel(x, token_table, pos_table):
    return _run(x.astype(jnp.int32), token_table, pos_table)


# speedup vs baseline: 6.9293x; 3.5428x over previous
import functools

import jax
import jax.numpy as jnp
from jax import lax
from jax.experimental import pallas as pl
from jax.experimental.pallas import tpu as pltpu
from jax.experimental.pallas import tpu_sc as plsc

B, L, V, D = 1024, 200, 100000, 128
NC, NS, LANES = 2, 16, 16
NW = NC * NS
SEQ_PER_W = B // NW
VECS_PER_ROW = D // LANES
C0 = 128
C1 = L - C0
NBUF = 3


def _sc_body(x_hbm, tok_hbm, pos_hbm, out_hbm,
             idx_all, pos_v, r0, r1, r2,
             gs0, gs1, gs2, os0, os1, os2, psem):
    rows = (r0, r1, r2)
    gsems = (gs0, gs1, gs2)
    osems = (os0, os1, os2)

    wid = lax.axis_index("s") * NC + lax.axis_index("c")
    seq0 = wid * SEQ_PER_W

    pos_cp = pltpu.async_copy(pos_hbm, pos_v, psem)
    pltpu.sync_copy(x_hbm.at[pl.ds(seq0, SEQ_PER_W)], idx_all)

    def start_gather(s):
        b = s % NBUF
        return (
            pltpu.async_copy(
                tok_hbm.at[idx_all.at[s, pl.ds(0, C0)]],
                rows[b].at[pl.ds(0, C0)], gsems[b]),
            pltpu.async_copy(
                tok_hbm.at[idx_all.at[s, pl.ds(C0, C1)]],
                rows[b].at[pl.ds(C0, C1)], gsems[b]),
        )

    gather_descs = {0: start_gather(0), 1: start_gather(1)}
    out_descs = {}

    pos_cp.wait()
    for s in range(SEQ_PER_W):
        b = s % NBUF
        for cp in gather_descs.pop(s):
            cp.wait()

        rows_b = rows[b]

        def add_body(r, c2, rows_b=rows_b):
            for c in range(VECS_PER_ROW):
                sl = pl.ds(c * LANES, LANES)
                plsc.addupdate(rows_b.at[r, sl], pos_v[r, sl])
            return c2

        lax.fori_loop(0, L, add_body, 0, unroll=4)

        out_descs[s] = pltpu.async_copy(rows_b, out_hbm.at[seq0 + s], osems[b])

        t = s + 2
        if t < SEQ_PER_W:
            if t >= NBUF:
                out_descs.pop(t - NBUF).wait()
            gather_descs[t] = start_gather(t)

    for s in sorted(out_descs):
        out_descs.pop(s).wait()


@jax.jit
def _run(x, token_table, pos_table):
    mesh = plsc.VectorSubcoreMesh(core_axis_name="c", subcore_axis_name="s")
    kfn = functools.partial(
        pl.kernel,
        mesh=mesh,
        out_type=jax.ShapeDtypeStruct((B, L, D), jnp.float32),
        scratch_types=[
            pltpu.VMEM((SEQ_PER_W, L), jnp.int32),
            pltpu.VMEM((L, D), jnp.float32),
            pltpu.VMEM((L, D), jnp.float32),
            pltpu.VMEM((L, D), jnp.float32),
            pltpu.VMEM((L, D), jnp.float32),
            pltpu.SemaphoreType.DMA,
            pltpu.SemaphoreType.DMA,
            pltpu.SemaphoreType.DMA,
            pltpu.SemaphoreType.DMA,
            pltpu.SemaphoreType.DMA,
            pltpu.SemaphoreType.DMA,
            pltpu.SemaphoreType.DMA,
        ],
    )(_sc_body)
    return kfn(x, token_table, pos_table)


def kernel(x, token_table, pos_table):
    return _run(x.astype(jnp.int32), token_table, pos_table)
